# Initial kernel scaffold; baseline (speedup 1.0000x reference)
import jax
import jax.numpy as jnp
from jax.experimental import pallas as pl


def kernel(users, edge_index, edge_values, user_table, item_table, w_ih, w_hh, b_ih, b_hh, W_gc, b_gc, h0):
    raise NotImplementedError("write your pallas kernel here")


# speedup vs baseline: 4.2105x; 4.2105x over previous
import functools

import jax
import jax.numpy as jnp
from jax import lax
from jax.experimental import pallas as pl
from jax.experimental.pallas import tpu as pltpu
from jax.experimental.pallas import tpu_sc as plsc

_NU = 10000
_NI = 40000
_NN = _NU + _NI
_D = 64
_DH = 32
_E = 800000
_NLAYERS = 3
_BETA = 0.001

_NC = 2
_NS = 16
_GRP = 128
_E_PAD = 802816
_NGRP = _E_PAD // _GRP
_GPT = _NGRP // _NS
_SUP = 56
_NSUP = _GPT // _SUP
_RPT = _NN // _NS
_ZR = 125


def _make_spmm():
    mesh = plsc.VectorSubcoreMesh(core_axis_name="c", subcore_axis_name="s")

    @functools.partial(
        pl.kernel,
        out_type=jax.ShapeDtypeStruct((_NN, _NC, _DH), jnp.float32),
        mesh=mesh,
        scratch_types=[
            pltpu.VMEM_SHARED((_NN, _DH), jnp.float32),
            pltpu.VMEM((_SUP, _GRP), jnp.int32),
            pltpu.VMEM((_SUP, _GRP), jnp.int32),
            pltpu.VMEM((_SUP, _GRP), jnp.float32),
            pltpu.VMEM((_GRP, _DH), jnp.float32),
            pltpu.VMEM((_ZR, _DH), jnp.float32),
            pltpu.SemaphoreType.DMA,
        ],
        compiler_params=pltpu.CompilerParams(use_tc_tiling_on_sc=False),
    )
    def spmm(table2, gidx, rows, vals, out, acc, idx_v, rows_v, vals_v,
             gath_v, zbuf, sem):
        cid = lax.axis_index("c")
        sid = lax.axis_index("s")

        zv = jnp.zeros((16,), jnp.float32)

        def zbuf_body(i, _):
            zbuf[i, pl.ds(0, 16)] = zv
            zbuf[i, pl.ds(16, 16)] = zv
            return 0

        lax.fori_loop(0, _ZR, zbuf_body, 0)
        rbase = sid * _RPT

        def zacc_body(k, _):
            pltpu.sync_copy(zbuf, acc.at[pl.ds(rbase + k * _ZR, _ZR)])
            return 0

        lax.fori_loop(0, _RPT // _ZR, zacc_body, 0)
        plsc.subcore_barrier()

        tile_g0 = sid * _GPT

        def super_body(sidx, _):
            g0 = tile_g0 + sidx * _SUP
            pltpu.sync_copy(gidx.at[cid, pl.ds(g0, _SUP)], idx_v)
            pltpu.sync_copy(rows.at[pl.ds(g0, _SUP)], rows_v)
            pltpu.sync_copy(vals.at[pl.ds(g0, _SUP)], vals_v)

            def grp_body(j, _):
                pltpu.async_copy(table2.at[idx_v.at[j]], gath_v, sem).wait()

                def scale_body(t, _):
                    base = t * 16
                    vv = vals_v[j, pl.ds(base, 16)]
                    for l in range(16):
                        v = vv[l]
                        e = base + l
                        gath_v[e, pl.ds(0, 16)] = gath_v[e, pl.ds(0, 16)] * v
                        gath_v[e, pl.ds(16, 16)] = gath_v[e, pl.ds(16, 16)] * v
                    return 0

                lax.fori_loop(0, _GRP // 16, scale_body, 0)
                pltpu.sync_copy(gath_v, acc.at[rows_v.at[j]], add=True)
                return 0

            lax.fori_loop(0, _SUP, grp_body, 0)
            return 0

        lax.fori_loop(0, _NSUP, super_body, 0)
        plsc.subcore_barrier()

        pltpu.sync_copy(acc.at[pl.ds(rbase, _RPT)],
                        out.at[pl.ds(rbase, _RPT), cid])

    return spmm


_make_spmm = functools.cache(_make_spmm)


def _make_user_gather():
    mesh = plsc.VectorSubcoreMesh(core_axis_name="c", subcore_axis_name="s")
    bpw = 1024 // (_NC * _NS)

    @functools.partial(
        pl.kernel,
        out_type=jax.ShapeDtypeStruct((1024, _D), jnp.float32),
        mesh=mesh,
        scratch_types=[
            pltpu.VMEM((bpw,), jnp.int32),
            pltpu.VMEM((bpw, _D), jnp.float32),
            pltpu.SemaphoreType.DMA,
        ],
        compiler_params=pltpu.CompilerParams(use_tc_tiling_on_sc=False),
    )
    def gather_k(table, idx, out, idx_v, rows_v, sem):
        wid = lax.axis_index("s") * _NC + lax.axis_index("c")
        base = wid * bpw
        pltpu.sync_copy(idx.at[pl.ds(base, bpw)], idx_v)
        pltpu.async_copy(table.at[idx_v], rows_v, sem).wait()
        pltpu.sync_copy(rows_v, out.at[pl.ds(base, bpw)])

    return gather_k


_make_user_gather = functools.cache(_make_user_gather)


_RB = 2000


def _dense_body(e_ref, h_ref, wih_ref, whh_ref, bih_ref, bhh_ref, wg_ref,
                bg_ref, out_ref):
    e = e_ref[...]
    h = h_ref[...]
    gi = jnp.dot(e, wih_ref[...], preferred_element_type=jnp.float32) + bih_ref[...]
    gh = jnp.dot(h, whh_ref[...], preferred_element_type=jnp.float32) + bhh_ref[...]
    i_r, i_z, i_n = gi[:, :_D], gi[:, _D:2 * _D], gi[:, 2 * _D:]
    h_r, h_z, h_n = gh[:, :_D], gh[:, _D:2 * _D], gh[:, 2 * _D:]
    r = jax.nn.sigmoid(i_r + h_r)
    z = jax.nn.sigmoid(i_z + h_z)
    n = jnp.tanh(i_n + r * h_n)
    gru = (1.0 - z) * n + z * h
    side = e * gru
    side = jnp.dot(side, wg_ref[...], preferred_element_type=jnp.float32) + bg_ref[...]
    x = side + e
    x = jnp.where(x >= 0.0, x, 0.2 * x)
    nrm = jnp.sqrt(jnp.sum(x * x, axis=1, keepdims=True))
    out_ref[...] = x / jnp.maximum(nrm, 1e-12)


def _dense_layer(e, h, wih_t, whh_t, bih, bhh, wg, bg):
    grid = _NN // _RB
    return pl.pallas_call(
        _dense_body,
        grid=(grid,),
        in_specs=[
            pl.BlockSpec((_RB, _D), lambda i: (i, 0)),
            pl.BlockSpec((_RB, _D), lambda i: (i, 0)),
            pl.BlockSpec((_D, 3 * _D), lambda i: (0, 0)),
            pl.BlockSpec((_D, 3 * _D), lambda i: (0, 0)),
            pl.BlockSpec((1, 3 * _D), lambda i: (0, 0)),
            pl.BlockSpec((1, 3 * _D), lambda i: (0, 0)),
            pl.BlockSpec((_D, _D), lambda i: (0, 0)),
            pl.BlockSpec((1, _D), lambda i: (0, 0)),
        ],
        out_specs=pl.BlockSpec((_RB, _D), lambda i: (i, 0)),
        out_shape=jax.ShapeDtypeStruct((_NN, _D), jnp.float32),
    )(e, h, wih_t, whh_t, bih, bhh, wg, bg)


def _combine_body(e0_ref, a1_ref, a2_ref, a3_ref, f1_ref, f2_ref, f3_ref,
                  out_ref):
    e0 = e0_ref[...]
    light1 = 0.25 * (e0 + a1_ref[...] + a2_ref[...] + a3_ref[...])
    light2 = e0 + f1_ref[...] + f2_ref[...] + f3_ref[...]
    out_ref[...] = light1 + _BETA * light2


def _combine(e0, a1, a2, a3, f1, f2, f3):
    grid = _NN // _RB
    spec = pl.BlockSpec((_RB, _D), lambda i: (i, 0))
    return pl.pallas_call(
        _combine_body,
        grid=(grid,),
        in_specs=[spec] * 7,
        out_specs=spec,
        out_shape=jax.ShapeDtypeStruct((_NN, _D), jnp.float32),
    )(e0, a1, a2, a3, f1, f2, f3)


_UB = 128


def _rating_body(u_ref, it_ref, out_ref):
    prod = lax.dot_general(u_ref[...], it_ref[...],
                           (((1,), (1,)), ((), ())),
                           preferred_element_type=jnp.float32)
    out_ref[...] = jax.nn.sigmoid(prod)


def _rating(u, items):
    grid = 1024 // _UB
    return pl.pallas_call(
        _rating_body,
        grid=(grid,),
        in_specs=[
            pl.BlockSpec((_UB, _D), lambda i: (i, 0)),
            pl.BlockSpec((_NI, _D), lambda i: (0, 0)),
        ],
        out_specs=pl.BlockSpec((_UB, _NI), lambda i: (i, 0)),
        out_shape=jax.ShapeDtypeStruct((1024, _NI), jnp.float32),
        compiler_params=pltpu.CompilerParams(vmem_limit_bytes=100 * 1024 * 1024),
    )(u, items)


def kernel(users, edge_index, edge_values, user_table, item_table,
           w_ih, w_hh, b_ih, b_hh, W_gc, b_gc, h0):
    row = edge_index[0]
    col = edge_index[1]
    pad = _E_PAD - _E
    col_p = jnp.concatenate([col, jnp.zeros((pad,), col.dtype)])
    row_p = jnp.concatenate([row, jnp.zeros((pad,), row.dtype)])
    val_p = jnp.concatenate([edge_values, jnp.zeros((pad,), edge_values.dtype)])
    gidx = jnp.stack([2 * col_p, 2 * col_p + 1]).reshape(_NC, _NGRP, _GRP)
    rows_g = row_p.reshape(_NGRP, _GRP)
    vals_g = val_p.reshape(_NGRP, _GRP)

    def spmm(emb):
        table2 = emb.reshape(_NN * _NC, _DH)
        out = _make_spmm()(table2, gidx, rows_g, vals_g)
        return out.reshape(_NN, _D)

    all_emb = jnp.concatenate([user_table, item_table], axis=0)

    a1 = spmm(all_emb)
    a2 = spmm(a1)
    a3 = spmm(a2)

    wih_t = w_ih.T
    whh_t = w_hh.T
    bih2 = b_ih.reshape(1, 3 * _D)
    bhh2 = b_hh.reshape(1, 3 * _D)
    f = all_emb
    fs = []
    for k in range(_NLAYERS):
        g = _dense_layer(f, h0[k, 0], wih_t, whh_t, bih2, bhh2,
                         W_gc[k], b_gc[k].reshape(1, _D))
        f = spmm(g)
        fs.append(f)

    combined = _combine(all_emb, a1, a2, a3, fs[0], fs[1], fs[2])

    u = _make_user_gather()(combined, users)
    items = combined[_NU:]
    return _rating(u, items)


# speedup vs baseline: 6.8941x; 1.6374x over previous
import functools

import jax
import jax.numpy as jnp
from jax import lax
from jax.experimental import pallas as pl
from jax.experimental.pallas import tpu as pltpu
from jax.experimental.pallas import tpu_sc as plsc

_NU = 10000
_NI = 40000
_NN = _NU + _NI
_D = 64
_DH = 32
_E = 800000
_NLAYERS = 3
_BETA = 0.001

_NC = 2
_NS = 16
_GRP = 128
_E_PAD = 802816
_NGRP = _E_PAD // _GRP
_GPT = _NGRP // _NS
_SUP = 8
_NSUP = _GPT // _SUP
_RPT = _NN // _NS
_ZR = 125


def _make_spmm():
    mesh = plsc.VectorSubcoreMesh(core_axis_name="c", subcore_axis_name="s")

    @functools.partial(
        pl.kernel,
        out_type=jax.ShapeDtypeStruct((_NN, _NC, _DH), jnp.float32),
        mesh=mesh,
        scratch_types=[
            pltpu.VMEM_SHARED((_NN, _DH), jnp.float32),
            pltpu.VMEM((2, _SUP, _GRP), jnp.int32),
            pltpu.VMEM((2, _SUP, _GRP), jnp.int32),
            pltpu.VMEM((2, _SUP, _GRP), jnp.float32),
            pltpu.VMEM((4, _GRP, _DH), jnp.float32),
            pltpu.VMEM((_ZR, _DH), jnp.float32),
            pltpu.SemaphoreType.DMA,
            pltpu.SemaphoreType.DMA,
            pltpu.SemaphoreType.DMA,
            pltpu.SemaphoreType.DMA,
            pltpu.SemaphoreType.DMA,
            pltpu.SemaphoreType.DMA,
            pltpu.SemaphoreType.DMA,
            pltpu.SemaphoreType.DMA,
            pltpu.SemaphoreType.DMA,
        ],
        compiler_params=pltpu.CompilerParams(use_tc_tiling_on_sc=False),
    )
    def spmm(table2, gidx, rows, vals, out, acc, idx_v, rows_v, vals_v,
             gath_v, zbuf, sg0, sg1, sg2, sg3, ss0, ss1, ss2, ss3, spf):
        cid = lax.axis_index("c")
        sid = lax.axis_index("s")

        zv = jnp.zeros((16,), jnp.float32)

        def zbuf_body(i, _):
            zbuf[i, pl.ds(0, 16)] = zv
            zbuf[i, pl.ds(16, 16)] = zv
            return 0

        lax.fori_loop(0, _ZR, zbuf_body, 0)
        rbase = sid * _RPT

        def zacc_body(k, _):
            pltpu.sync_copy(zbuf, acc.at[pl.ds(rbase + k * _ZR, _ZR)])
            return 0

        lax.fori_loop(0, _RPT // _ZR, zacc_body, 0)
        plsc.subcore_barrier()

        tile_g0 = sid * _GPT
        sgs = (sg0, sg1, sg2, sg3)
        sss = (ss0, ss1, ss2, ss3)

        def issue_gather(p, s, b):
            pltpu.async_copy(table2.at[idx_v.at[p, s]], gath_v.at[b], sgs[b])

        def wait_gather(b):
            pltpu.make_async_copy(table2.at[idx_v.at[0, 0]], gath_v.at[b],
                                  sgs[b]).wait()

        def issue_scatter(p, s, b):
            pltpu.async_copy(gath_v.at[b], acc.at[rows_v.at[p, s]], sss[b],
                             add=True)

        def wait_scatter(b):
            pltpu.make_async_copy(gath_v.at[b], acc.at[rows_v.at[0, 0]],
                                  sss[b]).wait()

        def issue_prefetch(p, g0):
            pltpu.async_copy(gidx.at[cid, pl.ds(g0, _SUP)], idx_v.at[p], spf)
            pltpu.async_copy(rows.at[pl.ds(g0, _SUP)], rows_v.at[p], spf)
            pltpu.async_copy(vals.at[pl.ds(g0, _SUP)], vals_v.at[p], spf)

        def wait_prefetch(p):
            pltpu.make_async_copy(gidx.at[cid, pl.ds(0, _SUP)],
                                  idx_v.at[p], spf).wait()
            pltpu.make_async_copy(rows.at[pl.ds(0, _SUP)],
                                  rows_v.at[p], spf).wait()
            pltpu.make_async_copy(vals.at[pl.ds(0, _SUP)],
                                  vals_v.at[p], spf).wait()

        def scale(p, s, b):
            def scale_body(t, _):
                base = t * 16
                vv = vals_v[p, s, pl.ds(base, 16)]
                for l in range(16):
                    v = vv[l]
                    e = base + l
                    gath_v[b, e, pl.ds(0, 16)] = gath_v[b, e, pl.ds(0, 16)] * v
                    gath_v[b, e, pl.ds(16, 16)] = gath_v[b, e, pl.ds(16, 16)] * v
                return 0

            lax.fori_loop(0, _GRP // 16, scale_body, 0)

        issue_prefetch(0, tile_g0)
        wait_prefetch(0)

        def super_body(sidx, _):
            p = lax.rem(sidx, 2)

            @pl.when(sidx + 1 < _NSUP)
            def _():
                issue_prefetch(1 - p, tile_g0 + (sidx + 1) * _SUP)

            issue_gather(p, 0, 0)
            issue_gather(p, 1, 1)

            def quad_body(q, _):
                for b in range(4):
                    s = q * 4 + b
                    bn = (b + 2) % 4

                    @pl.when(s >= 2)
                    def _():
                        wait_scatter(bn)

                    @pl.when(s + 2 < _SUP)
                    def _():
                        issue_gather(p, s + 2, bn)

                    wait_gather(b)
                    scale(p, s, b)
                    issue_scatter(p, s, b)
                return 0

            lax.fori_loop(0, _SUP // 4, quad_body, 0)
            wait_scatter((_SUP - 2) % 4)
            wait_scatter((_SUP - 1) % 4)

            @pl.when(sidx + 1 < _NSUP)
            def _():
                wait_prefetch(1 - p)

            return 0

        lax.fori_loop(0, _NSUP, super_body, 0)
        plsc.subcore_barrier()

        pltpu.sync_copy(acc.at[pl.ds(rbase, _RPT)],
                        out.at[pl.ds(rbase, _RPT), cid])

    return spmm


_make_spmm = functools.cache(_make_spmm)


def _make_user_gather():
    mesh = plsc.VectorSubcoreMesh(core_axis_name="c", subcore_axis_name="s")
    bpw = 1024 // (_NC * _NS)

    @functools.partial(
        pl.kernel,
        out_type=jax.ShapeDtypeStruct((1024, _D), jnp.float32),
        mesh=mesh,
        scratch_types=[
            pltpu.VMEM((bpw,), jnp.int32),
            pltpu.VMEM((bpw, _D), jnp.float32),
            pltpu.SemaphoreType.DMA,
        ],
        compiler_params=pltpu.CompilerParams(use_tc_tiling_on_sc=False),
    )
    def gather_k(table, idx, out, idx_v, rows_v, sem):
        wid = lax.axis_index("s") * _NC + lax.axis_index("c")
        base = wid * bpw
        pltpu.sync_copy(idx.at[pl.ds(base, bpw)], idx_v)
        pltpu.async_copy(table.at[idx_v], rows_v, sem).wait()
        pltpu.sync_copy(rows_v, out.at[pl.ds(base, bpw)])

    return gather_k


_make_user_gather = functools.cache(_make_user_gather)


_RB = 2000


def _dense_body(e_ref, h_ref, wih_ref, whh_ref, bih_ref, bhh_ref, wg_ref,
                bg_ref, out_ref):
    e = e_ref[...]
    h = h_ref[...]
    gi = jnp.dot(e, wih_ref[...], preferred_element_type=jnp.float32) + bih_ref[...]
    gh = jnp.dot(h, whh_ref[...], preferred_element_type=jnp.float32) + bhh_ref[...]
    i_r, i_z, i_n = gi[:, :_D], gi[:, _D:2 * _D], gi[:, 2 * _D:]
    h_r, h_z, h_n = gh[:, :_D], gh[:, _D:2 * _D], gh[:, 2 * _D:]
    r = jax.nn.sigmoid(i_r + h_r)
    z = jax.nn.sigmoid(i_z + h_z)
    n = jnp.tanh(i_n + r * h_n)
    gru = (1.0 - z) * n + z * h
    side = e * gru
    side = jnp.dot(side, wg_ref[...], preferred_element_type=jnp.float32) + bg_ref[...]
    x = side + e
    x = jnp.where(x >= 0.0, x, 0.2 * x)
    nrm = jnp.sqrt(jnp.sum(x * x, axis=1, keepdims=True))
    out_ref[...] = x / jnp.maximum(nrm, 1e-12)


def _dense_layer(e, h, wih_t, whh_t, bih, bhh, wg, bg):
    grid = _NN // _RB
    return pl.pallas_call(
        _dense_body,
        grid=(grid,),
        in_specs=[
            pl.BlockSpec((_RB, _D), lambda i: (i, 0)),
            pl.BlockSpec((_RB, _D), lambda i: (i, 0)),
            pl.BlockSpec((_D, 3 * _D), lambda i: (0, 0)),
            pl.BlockSpec((_D, 3 * _D), lambda i: (0, 0)),
            pl.BlockSpec((1, 3 * _D), lambda i: (0, 0)),
            pl.BlockSpec((1, 3 * _D), lambda i: (0, 0)),
            pl.BlockSpec((_D, _D), lambda i: (0, 0)),
            pl.BlockSpec((1, _D), lambda i: (0, 0)),
        ],
        out_specs=pl.BlockSpec((_RB, _D), lambda i: (i, 0)),
        out_shape=jax.ShapeDtypeStruct((_NN, _D), jnp.float32),
    )(e, h, wih_t, whh_t, bih, bhh, wg, bg)


def _combine_body(e0_ref, a1_ref, a2_ref, a3_ref, f1_ref, f2_ref, f3_ref,
                  out_ref):
    e0 = e0_ref[...]
    light1 = 0.25 * (e0 + a1_ref[...] + a2_ref[...] + a3_ref[...])
    light2 = e0 + f1_ref[...] + f2_ref[...] + f3_ref[...]
    out_ref[...] = light1 + _BETA * light2


def _combine(e0, a1, a2, a3, f1, f2, f3):
    grid = _NN // _RB
    spec = pl.BlockSpec((_RB, _D), lambda i: (i, 0))
    return pl.pallas_call(
        _combine_body,
        grid=(grid,),
        in_specs=[spec] * 7,
        out_specs=spec,
        out_shape=jax.ShapeDtypeStruct((_NN, _D), jnp.float32),
    )(e0, a1, a2, a3, f1, f2, f3)


_UB = 128


def _rating_body(u_ref, it_ref, out_ref):
    prod = lax.dot_general(u_ref[...], it_ref[...],
                           (((1,), (1,)), ((), ())),
                           preferred_element_type=jnp.float32)
    out_ref[...] = jax.nn.sigmoid(prod)


def _rating(u, items):
    grid = 1024 // _UB
    return pl.pallas_call(
        _rating_body,
        grid=(grid,),
        in_specs=[
            pl.BlockSpec((_UB, _D), lambda i: (i, 0)),
            pl.BlockSpec((_NI, _D), lambda i: (0, 0)),
        ],
        out_specs=pl.BlockSpec((_UB, _NI), lambda i: (i, 0)),
        out_shape=jax.ShapeDtypeStruct((1024, _NI), jnp.float32),
        compiler_params=pltpu.CompilerParams(vmem_limit_bytes=100 * 1024 * 1024),
    )(u, items)


def kernel(users, edge_index, edge_values, user_table, item_table,
           w_ih, w_hh, b_ih, b_hh, W_gc, b_gc, h0):
    row = edge_index[0]
    col = edge_index[1]
    pad = _E_PAD - _E
    col_p = jnp.concatenate([col, jnp.zeros((pad,), col.dtype)])
    row_p = jnp.concatenate([row, jnp.zeros((pad,), row.dtype)])
    val_p = jnp.concatenate([edge_values, jnp.zeros((pad,), edge_values.dtype)])
    gidx = jnp.stack([2 * col_p, 2 * col_p + 1]).reshape(_NC, _NGRP, _GRP)
    rows_g = row_p.reshape(_NGRP, _GRP)
    vals_g = val_p.reshape(_NGRP, _GRP)

    def spmm(emb):
        table2 = emb.reshape(_NN * _NC, _DH)
        out = _make_spmm()(table2, gidx, rows_g, vals_g)
        return out.reshape(_NN, _D)

    all_emb = jnp.concatenate([user_table, item_table], axis=0)

    a1 = spmm(all_emb)
    a2 = spmm(a1)
    a3 = spmm(a2)

    wih_t = w_ih.T
    whh_t = w_hh.T
    bih2 = b_ih.reshape(1, 3 * _D)
    bhh2 = b_hh.reshape(1, 3 * _D)
    f = all_emb
    fs = []
    for k in range(_NLAYERS):
        g = _dense_layer(f, h0[k, 0], wih_t, whh_t, bih2, bhh2,
                         W_gc[k], b_gc[k].reshape(1, _D))
        f = spmm(g)
        fs.append(f)

    combined = _combine(all_emb, a1, a2, a3, fs[0], fs[1], fs[2])

    u = _make_user_gather()(combined, users)
    items = combined[_NU:]
    return _rating(u, items)


# speedup vs baseline: 7.8413x; 1.1374x over previous
import functools

import jax
import jax.numpy as jnp
from jax import lax
from jax.experimental import pallas as pl
from jax.experimental.pallas import tpu as pltpu
from jax.experimental.pallas import tpu_sc as plsc

_NU = 10000
_NI = 40000
_NN = _NU + _NI
_D = 64
_DH = 32
_E = 800000
_NLAYERS = 3
_BETA = 0.001

_NC = 2
_NS = 16
_GRP = 128
_E_PAD = 802816
_NGRP = _E_PAD // _GRP
_GPT = _NGRP // _NS
_SUP = 8
_NSUP = _GPT // _SUP
_RPT = _NN // _NS
_ZR = 125


def _make_spmm():
    mesh = plsc.VectorSubcoreMesh(core_axis_name="c", subcore_axis_name="s")

    @functools.partial(
        pl.kernel,
        out_type=(jax.ShapeDtypeStruct((_NN, _DH), jnp.float32),
                  jax.ShapeDtypeStruct((_NN, _DH), jnp.float32)),
        mesh=mesh,
        scratch_types=[
            pltpu.VMEM_SHARED((_NN, _DH), jnp.float32),
            pltpu.VMEM((2, _SUP, _GRP), jnp.int32),
            pltpu.VMEM((2, _SUP, _GRP), jnp.int32),
            pltpu.VMEM((2, _SUP, _GRP), jnp.float32),
            pltpu.VMEM((4, _GRP, _DH), jnp.float32),
            pltpu.VMEM((_ZR, _DH), jnp.float32),
            pltpu.SemaphoreType.DMA,
            pltpu.SemaphoreType.DMA,
            pltpu.SemaphoreType.DMA,
            pltpu.SemaphoreType.DMA,
            pltpu.SemaphoreType.DMA,
            pltpu.SemaphoreType.DMA,
            pltpu.SemaphoreType.DMA,
            pltpu.SemaphoreType.DMA,
            pltpu.SemaphoreType.DMA,
        ],
        compiler_params=pltpu.CompilerParams(use_tc_tiling_on_sc=False),
    )
    def spmm(tlo, thi, cols, rows, vals, out_lo, out_hi, acc, idx_v, rows_v,
             vals_v, gath_v, zbuf, sg0, sg1, sg2, sg3, ss0, ss1, ss2, ss3,
             spf):
        cid = lax.axis_index("c")
        sid = lax.axis_index("s")

        zv = jnp.zeros((16,), jnp.float32)

        def zbuf_body(i, _):
            zbuf[i, pl.ds(0, 16)] = zv
            zbuf[i, pl.ds(16, 16)] = zv
            return 0

        lax.fori_loop(0, _ZR, zbuf_body, 0)
        rbase = sid * _RPT

        def zacc_body(k, _):
            pltpu.sync_copy(zbuf, acc.at[pl.ds(rbase + k * _ZR, _ZR)])
            return 0

        lax.fori_loop(0, _RPT // _ZR, zacc_body, 0)
        plsc.subcore_barrier()

        tile_g0 = sid * _GPT
        sgs = (sg0, sg1, sg2, sg3)
        sss = (ss0, ss1, ss2, ss3)

        def issue_gather(p, s, b):
            @pl.when(cid == 0)
            def _():
                pltpu.async_copy(tlo.at[idx_v.at[p, s]], gath_v.at[b], sgs[b])

            @pl.when(cid == 1)
            def _():
                pltpu.async_copy(thi.at[idx_v.at[p, s]], gath_v.at[b], sgs[b])

        def wait_gather(b):
            pltpu.make_async_copy(tlo.at[idx_v.at[0, 0]], gath_v.at[b],
                                  sgs[b]).wait()

        def issue_scatter(p, s, b):
            pltpu.async_copy(gath_v.at[b], acc.at[rows_v.at[p, s]], sss[b],
                             add=True)

        def wait_scatter(b):
            pltpu.make_async_copy(gath_v.at[b], acc.at[rows_v.at[0, 0]],
                                  sss[b]).wait()

        def issue_prefetch(p, g0):
            pltpu.async_copy(cols.at[pl.ds(g0, _SUP)], idx_v.at[p], spf)
            pltpu.async_copy(rows.at[pl.ds(g0, _SUP)], rows_v.at[p], spf)
            pltpu.async_copy(vals.at[pl.ds(g0, _SUP)], vals_v.at[p], spf)

        def wait_prefetch(p):
            pltpu.make_async_copy(cols.at[pl.ds(0, _SUP)],
                                  idx_v.at[p], spf).wait()
            pltpu.make_async_copy(rows.at[pl.ds(0, _SUP)],
                                  rows_v.at[p], spf).wait()
            pltpu.make_async_copy(vals.at[pl.ds(0, _SUP)],
                                  vals_v.at[p], spf).wait()

        def scale(p, s, b):
            def scale_body(t, _):
                base = t * 16
                vv = vals_v[p, s, pl.ds(base, 16)]
                for l in range(16):
                    v = vv[l]
                    e = base + l
                    gath_v[b, e, pl.ds(0, 16)] = gath_v[b, e, pl.ds(0, 16)] * v
                    gath_v[b, e, pl.ds(16, 16)] = gath_v[b, e, pl.ds(16, 16)] * v
                return 0

            lax.fori_loop(0, _GRP // 16, scale_body, 0)

        issue_prefetch(0, tile_g0)
        wait_prefetch(0)

        def super_body(sidx, _):
            p = lax.rem(sidx, 2)

            @pl.when(sidx + 1 < _NSUP)
            def _():
                issue_prefetch(1 - p, tile_g0 + (sidx + 1) * _SUP)

            issue_gather(p, 0, 0)
            issue_gather(p, 1, 1)

            def quad_body(q, _):
                for b in range(4):
                    s = q * 4 + b
                    bn = (b + 2) % 4

                    @pl.when(s >= 2)
                    def _():
                        wait_scatter(bn)

                    @pl.when(s + 2 < _SUP)
                    def _():
                        issue_gather(p, s + 2, bn)

                    wait_gather(b)
                    scale(p, s, b)
                    issue_scatter(p, s, b)
                return 0

            lax.fori_loop(0, _SUP // 4, quad_body, 0)
            wait_scatter((_SUP - 2) % 4)
            wait_scatter((_SUP - 1) % 4)

            @pl.when(sidx + 1 < _NSUP)
            def _():
                wait_prefetch(1 - p)

            return 0

        lax.fori_loop(0, _NSUP, super_body, 0)
        plsc.subcore_barrier()

        @pl.when(cid == 0)
        def _():
            pltpu.sync_copy(acc.at[pl.ds(rbase, _RPT)],
                            out_lo.at[pl.ds(rbase, _RPT)])

        @pl.when(cid == 1)
        def _():
            pltpu.sync_copy(acc.at[pl.ds(rbase, _RPT)],
                            out_hi.at[pl.ds(rbase, _RPT)])

    return spmm


_make_spmm = functools.cache(_make_spmm)


def _make_user_gather():
    mesh = plsc.VectorSubcoreMesh(core_axis_name="c", subcore_axis_name="s")
    bpw = 1024 // (_NC * _NS)

    @functools.partial(
        pl.kernel,
        out_type=jax.ShapeDtypeStruct((1024, _D), jnp.float32),
        mesh=mesh,
        scratch_types=[
            pltpu.VMEM((bpw,), jnp.int32),
            pltpu.VMEM((bpw, _D), jnp.float32),
            pltpu.SemaphoreType.DMA,
        ],
        compiler_params=pltpu.CompilerParams(use_tc_tiling_on_sc=False),
    )
    def gather_k(table, idx, out, idx_v, rows_v, sem):
        wid = lax.axis_index("s") * _NC + lax.axis_index("c")
        base = wid * bpw
        pltpu.sync_copy(idx.at[pl.ds(base, bpw)], idx_v)
        pltpu.async_copy(table.at[idx_v], rows_v, sem).wait()
        pltpu.sync_copy(rows_v, out.at[pl.ds(base, bpw)])

    return gather_k


_make_user_gather = functools.cache(_make_user_gather)


_RB = 2000


def _dense_body(elo_ref, ehi_ref, h_ref, wihlo_ref, wihhi_ref, whh_ref,
                bih_ref, bhh_ref, wglo_ref, wghi_ref, bg_ref, outlo_ref,
                outhi_ref):
    elo = elo_ref[...]
    ehi = ehi_ref[...]
    h = h_ref[...]
    gi = (jnp.dot(elo, wihlo_ref[...], preferred_element_type=jnp.float32)
          + jnp.dot(ehi, wihhi_ref[...], preferred_element_type=jnp.float32)
          + bih_ref[...])
    gh = jnp.dot(h, whh_ref[...], preferred_element_type=jnp.float32) + bhh_ref[...]
    r = jax.nn.sigmoid(gi[:, :_D] + gh[:, :_D])
    z = jax.nn.sigmoid(gi[:, _D:2 * _D] + gh[:, _D:2 * _D])
    n = jnp.tanh(gi[:, 2 * _D:] + r * gh[:, 2 * _D:])
    gru = (1.0 - z) * n + z * h
    sidelo = elo * gru[:, :_DH]
    sidehi = ehi * gru[:, _DH:]
    side = (jnp.dot(sidelo, wglo_ref[...], preferred_element_type=jnp.float32)
            + jnp.dot(sidehi, wghi_ref[...], preferred_element_type=jnp.float32)
            + bg_ref[...])
    xlo = side[:, :_DH] + elo
    xhi = side[:, _DH:] + ehi
    xlo = jnp.where(xlo >= 0.0, xlo, 0.2 * xlo)
    xhi = jnp.where(xhi >= 0.0, xhi, 0.2 * xhi)
    nrm = jnp.sqrt(jnp.sum(xlo * xlo, axis=1, keepdims=True)
                   + jnp.sum(xhi * xhi, axis=1, keepdims=True))
    nrm = jnp.maximum(nrm, 1e-12)
    outlo_ref[...] = xlo / nrm
    outhi_ref[...] = xhi / nrm


def _dense_layer(elo, ehi, h, wih_t_lo, wih_t_hi, whh_t, bih, bhh, wg_lo,
                 wg_hi, bg):
    grid = _NN // _RB
    return pl.pallas_call(
        _dense_body,
        grid=(grid,),
        in_specs=[
            pl.BlockSpec((_RB, _DH), lambda i: (i, 0)),
            pl.BlockSpec((_RB, _DH), lambda i: (i, 0)),
            pl.BlockSpec((_RB, _D), lambda i: (i, 0)),
            pl.BlockSpec((_DH, 3 * _D), lambda i: (0, 0)),
            pl.BlockSpec((_DH, 3 * _D), lambda i: (0, 0)),
            pl.BlockSpec((_D, 3 * _D), lambda i: (0, 0)),
            pl.BlockSpec((1, 3 * _D), lambda i: (0, 0)),
            pl.BlockSpec((1, 3 * _D), lambda i: (0, 0)),
            pl.BlockSpec((_DH, _D), lambda i: (0, 0)),
            pl.BlockSpec((_DH, _D), lambda i: (0, 0)),
            pl.BlockSpec((1, _D), lambda i: (0, 0)),
        ],
        out_specs=[
            pl.BlockSpec((_RB, _DH), lambda i: (i, 0)),
            pl.BlockSpec((_RB, _DH), lambda i: (i, 0)),
        ],
        out_shape=(jax.ShapeDtypeStruct((_NN, _DH), jnp.float32),
                   jax.ShapeDtypeStruct((_NN, _DH), jnp.float32)),
    )(elo, ehi, h, wih_t_lo, wih_t_hi, whh_t, bih, bhh, wg_lo, wg_hi, bg)


def _combine_body(*refs):
    out_ref = refs[-1]
    for half, base in ((0, 0), (1, 7)):
        e0 = refs[base][...]
        light1 = 0.25 * (e0 + refs[base + 1][...] + refs[base + 2][...]
                         + refs[base + 3][...])
        light2 = e0 + refs[base + 4][...] + refs[base + 5][...] + refs[base + 6][...]
        out_ref[:, pl.ds(half * _DH, _DH)] = light1 + _BETA * light2


def _combine(los, his):
    grid = _NN // _RB
    spec = pl.BlockSpec((_RB, _DH), lambda i: (i, 0))
    return pl.pallas_call(
        _combine_body,
        grid=(grid,),
        in_specs=[spec] * 14,
        out_specs=pl.BlockSpec((_RB, _D), lambda i: (i, 0)),
        out_shape=jax.ShapeDtypeStruct((_NN, _D), jnp.float32),
    )(*los, *his)


_UB = 128


def _rating_body(u_ref, it_ref, out_ref):
    prod = lax.dot_general(u_ref[...], it_ref[...],
                           (((1,), (1,)), ((), ())),
                           preferred_element_type=jnp.float32)
    out_ref[...] = jax.nn.sigmoid(prod)


def _rating(u, items):
    grid = 1024 // _UB
    return pl.pallas_call(
        _rating_body,
        grid=(grid,),
        in_specs=[
            pl.BlockSpec((_UB, _D), lambda i: (i, 0)),
            pl.BlockSpec((_NI, _D), lambda i: (0, 0)),
        ],
        out_specs=pl.BlockSpec((_UB, _NI), lambda i: (i, 0)),
        out_shape=jax.ShapeDtypeStruct((1024, _NI), jnp.float32),
        compiler_params=pltpu.CompilerParams(vmem_limit_bytes=100 * 1024 * 1024),
    )(u, items)


def kernel(users, edge_index, edge_values, user_table, item_table,
           w_ih, w_hh, b_ih, b_hh, W_gc, b_gc, h0):
    row = edge_index[0]
    col = edge_index[1]
    pad = _E_PAD - _E
    col_p = jnp.concatenate([col, jnp.zeros((pad,), col.dtype)])
    row_p = jnp.concatenate([row, jnp.zeros((pad,), row.dtype)])
    val_p = jnp.concatenate([edge_values, jnp.zeros((pad,), edge_values.dtype)])
    cols_g = col_p.reshape(_NGRP, _GRP)
    rows_g = row_p.reshape(_NGRP, _GRP)
    vals_g = val_p.reshape(_NGRP, _GRP)

    def spmm(emb_lo, emb_hi):
        return _make_spmm()(emb_lo, emb_hi, cols_g, rows_g, vals_g)

    e0_lo = jnp.concatenate([user_table[:, :_DH], item_table[:, :_DH]], axis=0)
    e0_hi = jnp.concatenate([user_table[:, _DH:], item_table[:, _DH:]], axis=0)

    a1 = spmm(e0_lo, e0_hi)
    a2 = spmm(*a1)
    a3 = spmm(*a2)

    wih_t = w_ih.T
    wih_t_lo = wih_t[:_DH]
    wih_t_hi = wih_t[_DH:]
    whh_t = w_hh.T
    bih2 = b_ih.reshape(1, 3 * _D)
    bhh2 = b_hh.reshape(1, 3 * _D)
    f_lo, f_hi = e0_lo, e0_hi
    fs = []
    for k in range(_NLAYERS):
        g_lo, g_hi = _dense_layer(f_lo, f_hi, h0[k, 0], wih_t_lo, wih_t_hi,
                                  whh_t, bih2, bhh2, W_gc[k][:_DH],
                                  W_gc[k][_DH:], b_gc[k].reshape(1, _D))
        f_lo, f_hi = spmm(g_lo, g_hi)
        fs.append((f_lo, f_hi))

    comb = _combine(
        (e0_lo, a1[0], a2[0], a3[0], fs[0][0], fs[1][0], fs[2][0]),
        (e0_hi, a1[1], a2[1], a3[1], fs[0][1], fs[1][1], fs[2][1]))

    u = _make_user_gather()(comb, users)
    return _rating(u, comb[_NU:])


# speedup vs baseline: 8.2875x; 1.0569x over previous
import functools

import jax
import jax.numpy as jnp
from jax import lax
from jax.experimental import pallas as pl
from jax.experimental.pallas import tpu as pltpu
from jax.experimental.pallas import tpu_sc as plsc

_NU = 10000
_NI = 40000
_NN = _NU + _NI
_D = 64
_DH = 32
_E = 800000
_NLAYERS = 3
_BETA = 0.001

_NC = 2
_NS = 16
_GRP = 128
_E_PAD = 802816
_NGRP = _E_PAD // _GRP
_GPT = _NGRP // _NS
_SUP = 8
_NSUP = _GPT // _SUP
_RPT = _NN // _NS
_ZR = 125


def _make_spmm(want_halves):
    mesh = plsc.VectorSubcoreMesh(core_axis_name="c", subcore_axis_name="s")
    outs = [jax.ShapeDtypeStruct((_NN, _D), jnp.float32)]
    if want_halves:
        outs += [jax.ShapeDtypeStruct((_NN, _DH), jnp.float32),
                 jax.ShapeDtypeStruct((_NN, _DH), jnp.float32)]

    @functools.partial(
        pl.kernel,
        out_type=tuple(outs),
        mesh=mesh,
        scratch_types=[
            pltpu.VMEM_SHARED((_NN, _DH), jnp.float32),
            pltpu.VMEM((2, _SUP, _GRP), jnp.int32),
            pltpu.VMEM((2, _SUP, _GRP), jnp.int32),
            pltpu.VMEM((2, _SUP, _GRP), jnp.float32),
            pltpu.VMEM((4, _GRP, _DH), jnp.float32),
            pltpu.VMEM((_ZR, _DH), jnp.float32),
            pltpu.SemaphoreType.DMA,
            pltpu.SemaphoreType.DMA,
            pltpu.SemaphoreType.DMA,
            pltpu.SemaphoreType.DMA,
            pltpu.SemaphoreType.DMA,
            pltpu.SemaphoreType.DMA,
            pltpu.SemaphoreType.DMA,
            pltpu.SemaphoreType.DMA,
            pltpu.SemaphoreType.DMA,
        ],
        compiler_params=pltpu.CompilerParams(use_tc_tiling_on_sc=False),
    )
    def spmm(tlo, thi, cols, rows, vals, *out_and_scratch):
        if want_halves:
            out64, out_lo, out_hi = out_and_scratch[:3]
            rest = out_and_scratch[3:]
        else:
            out64 = out_and_scratch[0]
            rest = out_and_scratch[1:]
        (acc, idx_v, rows_v, vals_v, gath_v, zbuf,
         sg0, sg1, sg2, sg3, ss0, ss1, ss2, ss3, spf) = rest
        cid = lax.axis_index("c")
        sid = lax.axis_index("s")

        zv = jnp.zeros((16,), jnp.float32)

        def zbuf_body(i, _):
            zbuf[i, pl.ds(0, 16)] = zv
            zbuf[i, pl.ds(16, 16)] = zv
            return 0

        lax.fori_loop(0, _ZR, zbuf_body, 0)
        rbase = sid * _RPT

        def zacc_body(k, _):
            pltpu.sync_copy(zbuf, acc.at[pl.ds(rbase + k * _ZR, _ZR)])
            return 0

        lax.fori_loop(0, _RPT // _ZR, zacc_body, 0)
        plsc.subcore_barrier()

        tile_g0 = sid * _GPT
        sgs = (sg0, sg1, sg2, sg3)
        sss = (ss0, ss1, ss2, ss3)

        def issue_gather(p, s, b):
            @pl.when(cid == 0)
            def _():
                pltpu.async_copy(tlo.at[idx_v.at[p, s]], gath_v.at[b], sgs[b])

            @pl.when(cid == 1)
            def _():
                pltpu.async_copy(thi.at[idx_v.at[p, s]], gath_v.at[b], sgs[b])

        def wait_gather(b):
            pltpu.make_async_copy(tlo.at[idx_v.at[0, 0]], gath_v.at[b],
                                  sgs[b]).wait()

        def issue_scatter(p, s, b):
            pltpu.async_copy(gath_v.at[b], acc.at[rows_v.at[p, s]], sss[b],
                             add=True)

        def wait_scatter(b):
            pltpu.make_async_copy(gath_v.at[b], acc.at[rows_v.at[0, 0]],
                                  sss[b]).wait()

        def issue_prefetch(p, g0):
            pltpu.async_copy(cols.at[pl.ds(g0, _SUP)], idx_v.at[p], spf)
            pltpu.async_copy(rows.at[pl.ds(g0, _SUP)], rows_v.at[p], spf)
            pltpu.async_copy(vals.at[pl.ds(g0, _SUP)], vals_v.at[p], spf)

        def wait_prefetch(p):
            pltpu.make_async_copy(cols.at[pl.ds(0, _SUP)],
                                  idx_v.at[p], spf).wait()
            pltpu.make_async_copy(rows.at[pl.ds(0, _SUP)],
                                  rows_v.at[p], spf).wait()
            pltpu.make_async_copy(vals.at[pl.ds(0, _SUP)],
                                  vals_v.at[p], spf).wait()

        def scale(p, s, b):
            def scale_body(t, _):
                base = t * 16
                vv = vals_v[p, s, pl.ds(base, 16)]
                for l in range(16):
                    v = vv[l]
                    e = base + l
                    gath_v[b, e, pl.ds(0, 16)] = gath_v[b, e, pl.ds(0, 16)] * v
                    gath_v[b, e, pl.ds(16, 16)] = gath_v[b, e, pl.ds(16, 16)] * v
                return 0

            lax.fori_loop(0, _GRP // 16, scale_body, 0)

        issue_prefetch(0, tile_g0)
        wait_prefetch(0)

        def super_body(sidx, _):
            p = lax.rem(sidx, 2)

            @pl.when(sidx + 1 < _NSUP)
            def _():
                issue_prefetch(1 - p, tile_g0 + (sidx + 1) * _SUP)

            issue_gather(p, 0, 0)
            issue_gather(p, 1, 1)

            def quad_body(q, _):
                for b in range(4):
                    s = q * 4 + b
                    bn = (b + 2) % 4

                    @pl.when(s >= 2)
                    def _():
                        wait_scatter(bn)

                    @pl.when(s + 2 < _SUP)
                    def _():
                        issue_gather(p, s + 2, bn)

                    wait_gather(b)
                    scale(p, s, b)
                    issue_scatter(p, s, b)
                return 0

            lax.fori_loop(0, _SUP // 4, quad_body, 0)
            wait_scatter((_SUP - 2) % 4)
            wait_scatter((_SUP - 1) % 4)

            @pl.when(sidx + 1 < _NSUP)
            def _():
                wait_prefetch(1 - p)

            return 0

        lax.fori_loop(0, _NSUP, super_body, 0)
        plsc.subcore_barrier()

        src = acc.at[pl.ds(rbase, _RPT)]

        @pl.when(cid == 0)
        def _():
            pltpu.sync_copy(src, out64.at[pl.ds(rbase, _RPT), pl.ds(0, _DH)])
            if want_halves:
                pltpu.sync_copy(src, out_lo.at[pl.ds(rbase, _RPT)])

        @pl.when(cid == 1)
        def _():
            pltpu.sync_copy(src, out64.at[pl.ds(rbase, _RPT), pl.ds(_DH, _DH)])
            if want_halves:
                pltpu.sync_copy(src, out_hi.at[pl.ds(rbase, _RPT)])

    return spmm


_make_spmm = functools.cache(_make_spmm)


def _make_user_gather():
    mesh = plsc.VectorSubcoreMesh(core_axis_name="c", subcore_axis_name="s")
    bpw = 1024 // (_NC * _NS)

    @functools.partial(
        pl.kernel,
        out_type=jax.ShapeDtypeStruct((1024, _D), jnp.float32),
        mesh=mesh,
        scratch_types=[
            pltpu.VMEM((bpw,), jnp.int32),
            pltpu.VMEM((bpw, _D), jnp.float32),
            pltpu.SemaphoreType.DMA,
        ],
        compiler_params=pltpu.CompilerParams(use_tc_tiling_on_sc=False),
    )
    def gather_k(table, idx, out, idx_v, rows_v, sem):
        wid = lax.axis_index("s") * _NC + lax.axis_index("c")
        base = wid * bpw
        pltpu.sync_copy(idx.at[pl.ds(base, bpw)], idx_v)
        pltpu.async_copy(table.at[idx_v], rows_v, sem).wait()
        pltpu.sync_copy(rows_v, out.at[pl.ds(base, bpw)])

    return gather_k


_make_user_gather = functools.cache(_make_user_gather)


_RB = 2000


def _split_body(ut_ref, it_ref, e64_ref, lo_ref, hi_ref):
    i = pl.program_id(0)
    x = jnp.where(jnp.full((1, 1), i < 5, jnp.bool_), ut_ref[...], it_ref[...])
    e64_ref[...] = x
    lo_ref[...] = x[:, :_DH]
    hi_ref[...] = x[:, _DH:]


def _split_tables(user_table, item_table):
    grid = _NN // _RB
    return pl.pallas_call(
        _split_body,
        grid=(grid,),
        in_specs=[
            pl.BlockSpec((_RB, _D), lambda i: (jnp.minimum(i, 4), 0)),
            pl.BlockSpec((_RB, _D), lambda i: (jnp.maximum(i - 5, 0), 0)),
        ],
        out_specs=[
            pl.BlockSpec((_RB, _D), lambda i: (i, 0)),
            pl.BlockSpec((_RB, _DH), lambda i: (i, 0)),
            pl.BlockSpec((_RB, _DH), lambda i: (i, 0)),
        ],
        out_shape=(jax.ShapeDtypeStruct((_NN, _D), jnp.float32),
                   jax.ShapeDtypeStruct((_NN, _DH), jnp.float32),
                   jax.ShapeDtypeStruct((_NN, _DH), jnp.float32)),
    )(user_table, item_table)


def _dense_body(e_ref, h_ref, wih_ref, whh_ref, bih_ref, bhh_ref, wg_ref,
                bg_ref, outlo_ref, outhi_ref):
    e = e_ref[...]
    h = h_ref[...]
    gi = jnp.dot(e, wih_ref[...], preferred_element_type=jnp.float32) + bih_ref[...]
    gh = jnp.dot(h, whh_ref[...], preferred_element_type=jnp.float32) + bhh_ref[...]
    r = jax.nn.sigmoid(gi[:, :_D] + gh[:, :_D])
    z = jax.nn.sigmoid(gi[:, _D:2 * _D] + gh[:, _D:2 * _D])
    n = jnp.tanh(gi[:, 2 * _D:] + r * gh[:, 2 * _D:])
    gru = (1.0 - z) * n + z * h
    side = e * gru
    side = jnp.dot(side, wg_ref[...], preferred_element_type=jnp.float32) + bg_ref[...]
    x = side + e
    x = jnp.where(x >= 0.0, x, 0.2 * x)
    nrm = jnp.sqrt(jnp.sum(x * x, axis=1, keepdims=True))
    x = x / jnp.maximum(nrm, 1e-12)
    outlo_ref[...] = x[:, :_DH]
    outhi_ref[...] = x[:, _DH:]


def _dense_layer(e, h, wih_t, whh_t, bih, bhh, wg, bg):
    grid = _NN // _RB
    return pl.pallas_call(
        _dense_body,
        grid=(grid,),
        in_specs=[
            pl.BlockSpec((_RB, _D), lambda i: (i, 0)),
            pl.BlockSpec((_RB, _D), lambda i: (i, 0)),
            pl.BlockSpec((_D, 3 * _D), lambda i: (0, 0)),
            pl.BlockSpec((_D, 3 * _D), lambda i: (0, 0)),
            pl.BlockSpec((1, 3 * _D), lambda i: (0, 0)),
            pl.BlockSpec((1, 3 * _D), lambda i: (0, 0)),
            pl.BlockSpec((_D, _D), lambda i: (0, 0)),
            pl.BlockSpec((1, _D), lambda i: (0, 0)),
        ],
        out_specs=[
            pl.BlockSpec((_RB, _DH), lambda i: (i, 0)),
            pl.BlockSpec((_RB, _DH), lambda i: (i, 0)),
        ],
        out_shape=(jax.ShapeDtypeStruct((_NN, _DH), jnp.float32),
                   jax.ShapeDtypeStruct((_NN, _DH), jnp.float32)),
    )(e, h, wih_t, whh_t, bih, bhh, wg, bg)


def _combine_body(e0_ref, a1_ref, a2_ref, a3_ref, f1_ref, f2_ref, f3_ref,
                  out_ref, items_ref):
    i = pl.program_id(0)
    e0 = e0_ref[...]
    light1 = 0.25 * (e0 + a1_ref[...] + a2_ref[...] + a3_ref[...])
    light2 = e0 + f1_ref[...] + f2_ref[...] + f3_ref[...]
    res = light1 + _BETA * light2
    out_ref[...] = res

    @pl.when(i >= 5)
    def _():
        items_ref[...] = res


def _combine(e0, a1, a2, a3, f1, f2, f3):
    grid = _NN // _RB
    spec = pl.BlockSpec((_RB, _D), lambda i: (i, 0))
    return pl.pallas_call(
        _combine_body,
        grid=(grid,),
        in_specs=[spec] * 7,
        out_specs=[
            spec,
            pl.BlockSpec((_RB, _D), lambda i: (jnp.maximum(i - 5, 0), 0)),
        ],
        out_shape=(jax.ShapeDtypeStruct((_NN, _D), jnp.float32),
                   jax.ShapeDtypeStruct((_NI, _D), jnp.float32)),
    )(e0, a1, a2, a3, f1, f2, f3)


_UB = 128


def _rating_body(u_ref, it_ref, out_ref):
    prod = lax.dot_general(u_ref[...], it_ref[...],
                           (((1,), (1,)), ((), ())),
                           preferred_element_type=jnp.float32)
    out_ref[...] = jax.nn.sigmoid(prod)


def _rating(u, items):
    grid = 1024 // _UB
    return pl.pallas_call(
        _rating_body,
        grid=(grid,),
        in_specs=[
            pl.BlockSpec((_UB, _D), lambda i: (i, 0)),
            pl.BlockSpec((_NI, _D), lambda i: (0, 0)),
        ],
        out_specs=pl.BlockSpec((_UB, _NI), lambda i: (i, 0)),
        out_shape=jax.ShapeDtypeStruct((1024, _NI), jnp.float32),
        compiler_params=pltpu.CompilerParams(vmem_limit_bytes=100 * 1024 * 1024),
    )(u, items)


def kernel(users, edge_index, edge_values, user_table, item_table,
           w_ih, w_hh, b_ih, b_hh, W_gc, b_gc, h0):
    row = edge_index[0]
    col = edge_index[1]
    pad = _E_PAD - _E
    col_p = jnp.concatenate([col, jnp.zeros((pad,), col.dtype)])
    row_p = jnp.concatenate([row, jnp.zeros((pad,), row.dtype)])
    val_p = jnp.concatenate([edge_values, jnp.zeros((pad,), edge_values.dtype)])
    cols_g = col_p.reshape(_NGRP, _GRP)
    rows_g = row_p.reshape(_NGRP, _GRP)
    vals_g = val_p.reshape(_NGRP, _GRP)

    def spmm(emb_lo, emb_hi, want_halves):
        return _make_spmm(want_halves)(emb_lo, emb_hi, cols_g, rows_g, vals_g)

    e0_64, e0_lo, e0_hi = _split_tables(user_table, item_table)

    a1_64, a1_lo, a1_hi = spmm(e0_lo, e0_hi, True)
    a2_64, a2_lo, a2_hi = spmm(a1_lo, a1_hi, True)
    (a3_64,) = spmm(a2_lo, a2_hi, False)

    wih_t = w_ih.T
    whh_t = w_hh.T
    bih2 = b_ih.reshape(1, 3 * _D)
    bhh2 = b_hh.reshape(1, 3 * _D)
    f64 = e0_64
    f64s = []
    for k in range(_NLAYERS):
        g_lo, g_hi = _dense_layer(f64, h0[k, 0], wih_t, whh_t, bih2, bhh2,
                                  W_gc[k], b_gc[k].reshape(1, _D))
        (f64,) = spmm(g_lo, g_hi, False)
        f64s.append(f64)

    comb, items = _combine(e0_64, a1_64, a2_64, a3_64, *f64s)

    u = _make_user_gather()(comb, users)
    return _rating(u, items)


# speedup vs baseline: 8.8688x; 1.0702x over previous
import functools

import jax
import jax.numpy as jnp
from jax import lax
from jax.experimental import pallas as pl
from jax.experimental.pallas import tpu as pltpu
from jax.experimental.pallas import tpu_sc as plsc

_NU = 10000
_NI = 40000
_NN = _NU + _NI
_D = 64
_DH = 32
_E = 800000
_NLAYERS = 3
_BETA = 0.001

_NC = 2
_NS = 16
_GRP = 128
_E_PAD = 802816
_NGRP = _E_PAD // _GRP
_GPT = _NGRP // _NS
_SUP = 8
_NSUP = _GPT // _SUP
_RPT = _NN // _NS
_ZR = 125


def _make_spmm(want_halves):
    mesh = plsc.VectorSubcoreMesh(core_axis_name="c", subcore_axis_name="s")
    outs = [jax.ShapeDtypeStruct((_NN, _D), jnp.float32)]
    if want_halves:
        outs += [jax.ShapeDtypeStruct((_NN, _DH), jnp.float32),
                 jax.ShapeDtypeStruct((_NN, _DH), jnp.float32)]

    @functools.partial(
        pl.kernel,
        out_type=tuple(outs),
        mesh=mesh,
        scratch_types=[
            pltpu.VMEM_SHARED((_NN, _DH), jnp.float32),
            pltpu.VMEM((2, _SUP, _GRP), jnp.int32),
            pltpu.VMEM((2, _SUP, _GRP), jnp.int32),
            pltpu.VMEM((2, _SUP, _GRP), jnp.float32),
            pltpu.VMEM((4, _GRP, _DH), jnp.float32),
            pltpu.VMEM((_ZR, _DH), jnp.float32),
            pltpu.SemaphoreType.DMA,
            pltpu.SemaphoreType.DMA,
            pltpu.SemaphoreType.DMA,
            pltpu.SemaphoreType.DMA,
            pltpu.SemaphoreType.DMA,
            pltpu.SemaphoreType.DMA,
            pltpu.SemaphoreType.DMA,
            pltpu.SemaphoreType.DMA,
            pltpu.SemaphoreType.DMA,
        ],
        compiler_params=pltpu.CompilerParams(use_tc_tiling_on_sc=False),
    )
    def spmm(tlo, thi, cols, rows, vals, *out_and_scratch):
        if want_halves:
            out64, out_lo, out_hi = out_and_scratch[:3]
            rest = out_and_scratch[3:]
        else:
            out64 = out_and_scratch[0]
            rest = out_and_scratch[1:]
        (acc, idx_v, rows_v, vals_v, gath_v, zbuf,
         sg0, sg1, sg2, sg3, ss0, ss1, ss2, ss3, spf) = rest
        cid = lax.axis_index("c")
        sid = lax.axis_index("s")

        zv = jnp.zeros((16,), jnp.float32)

        def zbuf_body(i, _):
            zbuf[i, pl.ds(0, 16)] = zv
            zbuf[i, pl.ds(16, 16)] = zv
            return 0

        lax.fori_loop(0, _ZR, zbuf_body, 0)
        rbase = sid * _RPT

        def zacc_body(k, _):
            pltpu.sync_copy(zbuf, acc.at[pl.ds(rbase + k * _ZR, _ZR)])
            return 0

        lax.fori_loop(0, _RPT // _ZR, zacc_body, 0)
        plsc.subcore_barrier()

        tile_g0 = sid * _GPT
        sgs = (sg0, sg1, sg2, sg3)
        sss = (ss0, ss1, ss2, ss3)

        def issue_gather(p, s, b):
            @pl.when(cid == 0)
            def _():
                pltpu.async_copy(tlo.at[idx_v.at[p, s]], gath_v.at[b], sgs[b])

            @pl.when(cid == 1)
            def _():
                pltpu.async_copy(thi.at[idx_v.at[p, s]], gath_v.at[b], sgs[b])

        def wait_gather(b):
            pltpu.make_async_copy(tlo.at[idx_v.at[0, 0]], gath_v.at[b],
                                  sgs[b]).wait()

        def issue_scatter(p, s, b):
            pltpu.async_copy(gath_v.at[b], acc.at[rows_v.at[p, s]], sss[b],
                             add=True)

        def wait_scatter(b):
            pltpu.make_async_copy(gath_v.at[b], acc.at[rows_v.at[0, 0]],
                                  sss[b]).wait()

        def issue_prefetch(p, g0):
            pltpu.async_copy(cols.at[pl.ds(g0, _SUP)], idx_v.at[p], spf)
            pltpu.async_copy(rows.at[pl.ds(g0, _SUP)], rows_v.at[p], spf)
            pltpu.async_copy(vals.at[pl.ds(g0, _SUP)], vals_v.at[p], spf)

        def wait_prefetch(p):
            pltpu.make_async_copy(cols.at[pl.ds(0, _SUP)],
                                  idx_v.at[p], spf).wait()
            pltpu.make_async_copy(rows.at[pl.ds(0, _SUP)],
                                  rows_v.at[p], spf).wait()
            pltpu.make_async_copy(vals.at[pl.ds(0, _SUP)],
                                  vals_v.at[p], spf).wait()

        def scale(p, s, b):
            for t in range(_GRP // 16):
                base = t * 16
                vv = vals_v[p, s, pl.ds(base, 16)]
                for l in range(16):
                    v = vv[l]
                    e = base + l
                    gath_v[b, e, pl.ds(0, 16)] = gath_v[b, e, pl.ds(0, 16)] * v
                    gath_v[b, e, pl.ds(16, 16)] = gath_v[b, e, pl.ds(16, 16)] * v

        issue_prefetch(0, tile_g0)
        wait_prefetch(0)

        def super_body(sidx, _):
            p = lax.rem(sidx, 2)

            @pl.when(sidx + 1 < _NSUP)
            def _():
                issue_prefetch(1 - p, tile_g0 + (sidx + 1) * _SUP)

            issue_gather(p, 0, 0)
            issue_gather(p, 1, 1)

            def quad_body(q, _):
                for b in range(4):
                    s = q * 4 + b
                    bn = (b + 2) % 4

                    @pl.when(s >= 2)
                    def _():
                        wait_scatter(bn)

                    @pl.when(s + 2 < _SUP)
                    def _():
                        issue_gather(p, s + 2, bn)

                    wait_gather(b)
                    scale(p, s, b)
                    issue_scatter(p, s, b)
                return 0

            lax.fori_loop(0, _SUP // 4, quad_body, 0)
            wait_scatter((_SUP - 2) % 4)
            wait_scatter((_SUP - 1) % 4)

            @pl.when(sidx + 1 < _NSUP)
            def _():
                wait_prefetch(1 - p)

            return 0

        lax.fori_loop(0, _NSUP, super_body, 0)
        plsc.subcore_barrier()

        src = acc.at[pl.ds(rbase, _RPT)]

        @pl.when(cid == 0)
        def _():
            pltpu.sync_copy(src, out64.at[pl.ds(rbase, _RPT), pl.ds(0, _DH)])
            if want_halves:
                pltpu.sync_copy(src, out_lo.at[pl.ds(rbase, _RPT)])

        @pl.when(cid == 1)
        def _():
            pltpu.sync_copy(src, out64.at[pl.ds(rbase, _RPT), pl.ds(_DH, _DH)])
            if want_halves:
                pltpu.sync_copy(src, out_hi.at[pl.ds(rbase, _RPT)])

    return spmm


_make_spmm = functools.cache(_make_spmm)


def _make_user_gather():
    mesh = plsc.VectorSubcoreMesh(core_axis_name="c", subcore_axis_name="s")
    bpw = 1024 // (_NC * _NS)

    @functools.partial(
        pl.kernel,
        out_type=jax.ShapeDtypeStruct((1024, _D), jnp.float32),
        mesh=mesh,
        scratch_types=[
            pltpu.VMEM((bpw,), jnp.int32),
            pltpu.VMEM((bpw, _D), jnp.float32),
            pltpu.SemaphoreType.DMA,
        ],
        compiler_params=pltpu.CompilerParams(use_tc_tiling_on_sc=False),
    )
    def gather_k(table, idx, out, idx_v, rows_v, sem):
        wid = lax.axis_index("s") * _NC + lax.axis_index("c")
        base = wid * bpw
        pltpu.sync_copy(idx.at[pl.ds(base, bpw)], idx_v)
        pltpu.async_copy(table.at[idx_v], rows_v, sem).wait()
        pltpu.sync_copy(rows_v, out.at[pl.ds(base, bpw)])

    return gather_k


_make_user_gather = functools.cache(_make_user_gather)


_RB = 2000


def _split_body(ut_ref, it_ref, e64_ref, lo_ref, hi_ref):
    i = pl.program_id(0)
    x = jnp.where(jnp.full((1, 1), i < 5, jnp.bool_), ut_ref[...], it_ref[...])
    e64_ref[...] = x
    lo_ref[...] = x[:, :_DH]
    hi_ref[...] = x[:, _DH:]


def _split_tables(user_table, item_table):
    grid = _NN // _RB
    return pl.pallas_call(
        _split_body,
        grid=(grid,),
        in_specs=[
            pl.BlockSpec((_RB, _D), lambda i: (jnp.minimum(i, 4), 0)),
            pl.BlockSpec((_RB, _D), lambda i: (jnp.maximum(i - 5, 0), 0)),
        ],
        out_specs=[
            pl.BlockSpec((_RB, _D), lambda i: (i, 0)),
            pl.BlockSpec((_RB, _DH), lambda i: (i, 0)),
            pl.BlockSpec((_RB, _DH), lambda i: (i, 0)),
        ],
        out_shape=(jax.ShapeDtypeStruct((_NN, _D), jnp.float32),
                   jax.ShapeDtypeStruct((_NN, _DH), jnp.float32),
                   jax.ShapeDtypeStruct((_NN, _DH), jnp.float32)),
    )(user_table, item_table)


def _dense_body(e_ref, h_ref, wih_ref, whh_ref, bih_ref, bhh_ref, wg_ref,
                bg_ref, outlo_ref, outhi_ref):
    e = e_ref[...]
    h = h_ref[0, 0]
    gi = jnp.dot(e, wih_ref[...], preferred_element_type=jnp.float32) + bih_ref[...]
    gh = jnp.dot(h, whh_ref[...], preferred_element_type=jnp.float32) + bhh_ref[...]
    r = jax.nn.sigmoid(gi[:, :_D] + gh[:, :_D])
    z = jax.nn.sigmoid(gi[:, _D:2 * _D] + gh[:, _D:2 * _D])
    n = jnp.tanh(gi[:, 2 * _D:] + r * gh[:, 2 * _D:])
    gru = (1.0 - z) * n + z * h
    side = e * gru
    side = jnp.dot(side, wg_ref[...], preferred_element_type=jnp.float32) + bg_ref[...]
    x = side + e
    x = jnp.where(x >= 0.0, x, 0.2 * x)
    nrm = jnp.sqrt(jnp.sum(x * x, axis=1, keepdims=True))
    x = x / jnp.maximum(nrm, 1e-12)
    outlo_ref[...] = x[:, :_DH]
    outhi_ref[...] = x[:, _DH:]


def _dense_layer(e, h0, k, wih_t, whh_t, bih, bhh, wg, bg):
    grid = _NN // _RB
    return pl.pallas_call(
        _dense_body,
        grid=(grid,),
        in_specs=[
            pl.BlockSpec((_RB, _D), lambda i: (i, 0)),
            pl.BlockSpec((1, 1, _RB, _D), lambda i, k=k: (k, 0, i, 0)),
            pl.BlockSpec((_D, 3 * _D), lambda i: (0, 0)),
            pl.BlockSpec((_D, 3 * _D), lambda i: (0, 0)),
            pl.BlockSpec((1, 3 * _D), lambda i: (0, 0)),
            pl.BlockSpec((1, 3 * _D), lambda i: (0, 0)),
            pl.BlockSpec((_D, _D), lambda i: (0, 0)),
            pl.BlockSpec((1, _D), lambda i: (0, 0)),
        ],
        out_specs=[
            pl.BlockSpec((_RB, _DH), lambda i: (i, 0)),
            pl.BlockSpec((_RB, _DH), lambda i: (i, 0)),
        ],
        out_shape=(jax.ShapeDtypeStruct((_NN, _DH), jnp.float32),
                   jax.ShapeDtypeStruct((_NN, _DH), jnp.float32)),
    )(e, h0, wih_t, whh_t, bih, bhh, wg, bg)


def _combine_body(e0_ref, a1_ref, a2_ref, a3_ref, f1_ref, f2_ref, f3_ref,
                  out_ref, items_ref):
    i = pl.program_id(0)
    e0 = e0_ref[...]
    light1 = 0.25 * (e0 + a1_ref[...] + a2_ref[...] + a3_ref[...])
    light2 = e0 + f1_ref[...] + f2_ref[...] + f3_ref[...]
    res = light1 + _BETA * light2
    out_ref[...] = res

    @pl.when(i >= 5)
    def _():
        items_ref[...] = res


def _combine(e0, a1, a2, a3, f1, f2, f3):
    grid = _NN // _RB
    spec = pl.BlockSpec((_RB, _D), lambda i: (i, 0))
    return pl.pallas_call(
        _combine_body,
        grid=(grid,),
        in_specs=[spec] * 7,
        out_specs=[
            spec,
            pl.BlockSpec((_RB, _D), lambda i: (jnp.maximum(i - 5, 0), 0)),
        ],
        out_shape=(jax.ShapeDtypeStruct((_NN, _D), jnp.float32),
                   jax.ShapeDtypeStruct((_NI, _D), jnp.float32)),
    )(e0, a1, a2, a3, f1, f2, f3)


_IB = 2000


def _rating_body(it_ref, u_ref, out_ref):
    prod = lax.dot_general(it_ref[...], u_ref[...],
                           (((1,), (1,)), ((), ())),
                           preferred_element_type=jnp.float32)
    out_ref[...] = jax.nn.sigmoid(prod)


def _rating(u, items):
    grid = _NI // _IB
    out_t = pl.pallas_call(
        _rating_body,
        grid=(grid,),
        in_specs=[
            pl.BlockSpec((_IB, _D), lambda i: (i, 0)),
            pl.BlockSpec((1024, _D), lambda i: (0, 0)),
        ],
        out_specs=pl.BlockSpec((_IB, 1024), lambda i: (i, 0)),
        out_shape=jax.ShapeDtypeStruct((_NI, 1024), jnp.float32),
    )(items, u)
    return out_t.T


def kernel(users, edge_index, edge_values, user_table, item_table,
           w_ih, w_hh, b_ih, b_hh, W_gc, b_gc, h0):
    row = edge_index[0]
    col = edge_index[1]
    pad = _E_PAD - _E
    col_p = jnp.concatenate([col, jnp.zeros((pad,), col.dtype)])
    row_p = jnp.concatenate([row, jnp.zeros((pad,), row.dtype)])
    val_p = jnp.concatenate([edge_values, jnp.zeros((pad,), edge_values.dtype)])
    cols_g = col_p.reshape(_NGRP, _GRP)
    rows_g = row_p.reshape(_NGRP, _GRP)
    vals_g = val_p.reshape(_NGRP, _GRP)

    def spmm(emb_lo, emb_hi, want_halves):
        return _make_spmm(want_halves)(emb_lo, emb_hi, cols_g, rows_g, vals_g)

    e0_64, e0_lo, e0_hi = _split_tables(user_table, item_table)

    a1_64, a1_lo, a1_hi = spmm(e0_lo, e0_hi, True)
    a2_64, a2_lo, a2_hi = spmm(a1_lo, a1_hi, True)
    (a3_64,) = spmm(a2_lo, a2_hi, False)

    wih_t = w_ih.T
    whh_t = w_hh.T
    bih2 = b_ih.reshape(1, 3 * _D)
    bhh2 = b_hh.reshape(1, 3 * _D)
    f64 = e0_64
    f64s = []
    for k in range(_NLAYERS):
        g_lo, g_hi = _dense_layer(f64, h0, k, wih_t, whh_t, bih2, bhh2,
                                  W_gc[k], b_gc[k].reshape(1, _D))
        (f64,) = spmm(g_lo, g_hi, False)
        f64s.append(f64)

    comb, items = _combine(e0_64, a1_64, a2_64, a3_64, *f64s)

    u = _make_user_gather()(comb, users)
    return _rating(u, items)


# speedup vs baseline: 9.2674x; 1.0449x over previous
import functools

import jax
import jax.numpy as jnp
from jax import lax
from jax.experimental import pallas as pl
from jax.experimental.pallas import tpu as pltpu
from jax.experimental.pallas import tpu_sc as plsc

_NU = 10000
_NI = 40000
_NN = _NU + _NI
_D = 64
_DH = 32
_E = 800000
_NLAYERS = 3
_BETA = 0.001

_NC = 2
_NS = 16
_GRP = 128
_E_PAD = 802816
_NGRP = _E_PAD // _GRP
_GPT = _NGRP // _NS
_SUP = 8
_NSUP = _GPT // _SUP
_RPT = _NN // _NS
_ZR = 125


def _make_spmm(want_halves):
    mesh = plsc.VectorSubcoreMesh(core_axis_name="c", subcore_axis_name="s")
    outs = [jax.ShapeDtypeStruct((_NN, _D), jnp.float32)]
    if want_halves:
        outs += [jax.ShapeDtypeStruct((_NN, _DH), jnp.float32),
                 jax.ShapeDtypeStruct((_NN, _DH), jnp.float32)]

    @functools.partial(
        pl.kernel,
        out_type=tuple(outs),
        mesh=mesh,
        scratch_types=[
            pltpu.VMEM_SHARED((_NN, _DH), jnp.float32),
            pltpu.VMEM((2, _SUP, _GRP), jnp.int32),
            pltpu.VMEM((2, _SUP, _GRP), jnp.int32),
            pltpu.VMEM((2, _SUP, _GRP), jnp.float32),
            pltpu.VMEM((5, _GRP, _DH), jnp.float32),
            pltpu.SemaphoreType.DMA,
            pltpu.SemaphoreType.DMA,
            pltpu.SemaphoreType.DMA,
            pltpu.SemaphoreType.DMA,
            pltpu.SemaphoreType.DMA,
            pltpu.SemaphoreType.DMA,
            pltpu.SemaphoreType.DMA,
            pltpu.SemaphoreType.DMA,
            pltpu.SemaphoreType.DMA,
            pltpu.SemaphoreType.DMA,
            pltpu.SemaphoreType.DMA,
        ],
        compiler_params=pltpu.CompilerParams(use_tc_tiling_on_sc=False),
    )
    def spmm(tlo, thi, cols, rows, vals, *out_and_scratch):
        if want_halves:
            out64, out_lo, out_hi = out_and_scratch[:3]
            rest = out_and_scratch[3:]
        else:
            out64 = out_and_scratch[0]
            rest = out_and_scratch[1:]
        (acc, idx_v, rows_v, vals_v, gath_v,
         sg0, sg1, sg2, sg3, sg4, ss0, ss1, ss2, ss3, ss4, spf) = rest
        cid = lax.axis_index("c")
        sid = lax.axis_index("s")

        zv = jnp.zeros((16,), jnp.float32)

        def zbuf_body(i, _):
            gath_v[0, i, pl.ds(0, 16)] = zv
            gath_v[0, i, pl.ds(16, 16)] = zv
            return 0

        lax.fori_loop(0, _GRP, zbuf_body, 0)
        rbase = sid * _RPT

        def zacc_body(k, _):
            pltpu.sync_copy(gath_v.at[0], acc.at[pl.ds(rbase + k * _GRP, _GRP)])
            return 0

        lax.fori_loop(0, _RPT // _GRP, zacc_body, 0)
        pltpu.sync_copy(gath_v.at[0, pl.ds(0, _RPT % _GRP)],
                        acc.at[pl.ds(rbase + _RPT - _RPT % _GRP, _RPT % _GRP)])
        plsc.subcore_barrier()

        tile_g0 = sid * _GPT
        sgs = (sg0, sg1, sg2, sg3, sg4)
        sss = (ss0, ss1, ss2, ss3, ss4)

        def issue_gather(p, s, b):
            @pl.when(cid == 0)
            def _():
                pltpu.async_copy(tlo.at[idx_v.at[p, s]], gath_v.at[b], sgs[b])

            @pl.when(cid == 1)
            def _():
                pltpu.async_copy(thi.at[idx_v.at[p, s]], gath_v.at[b], sgs[b])

        def wait_gather(b):
            pltpu.make_async_copy(tlo.at[idx_v.at[0, 0]], gath_v.at[b],
                                  sgs[b]).wait()

        def issue_scatter(p, s, b):
            pltpu.async_copy(gath_v.at[b], acc.at[rows_v.at[p, s]], sss[b],
                             add=True)

        def wait_scatter(b):
            pltpu.make_async_copy(gath_v.at[b], acc.at[rows_v.at[0, 0]],
                                  sss[b]).wait()

        def issue_prefetch(p, g0):
            pltpu.async_copy(cols.at[pl.ds(g0, _SUP)], idx_v.at[p], spf)
            pltpu.async_copy(rows.at[pl.ds(g0, _SUP)], rows_v.at[p], spf)
            pltpu.async_copy(vals.at[pl.ds(g0, _SUP)], vals_v.at[p], spf)

        def wait_prefetch(p):
            pltpu.make_async_copy(cols.at[pl.ds(0, _SUP)],
                                  idx_v.at[p], spf).wait()
            pltpu.make_async_copy(rows.at[pl.ds(0, _SUP)],
                                  rows_v.at[p], spf).wait()
            pltpu.make_async_copy(vals.at[pl.ds(0, _SUP)],
                                  vals_v.at[p], spf).wait()

        def scale(p, s, b):
            def scale_body(t, _):
                base = t * 16
                vv = vals_v[p, s, pl.ds(base, 16)]
                for l in range(16):
                    v = vv[l]
                    e = base + l
                    gath_v[b, e, pl.ds(0, 16)] = gath_v[b, e, pl.ds(0, 16)] * v
                    gath_v[b, e, pl.ds(16, 16)] = gath_v[b, e, pl.ds(16, 16)] * v
                return 0

            lax.fori_loop(0, _GRP // 16, scale_body, 0)

        issue_prefetch(0, tile_g0)
        wait_prefetch(0)

        def super_body(sidx, _):
            p = lax.rem(sidx, 2)

            @pl.when(sidx + 1 < _NSUP)
            def _():
                issue_prefetch(1 - p, tile_g0 + (sidx + 1) * _SUP)

            issue_gather(p, 0, 0)
            issue_gather(p, 1, 1)
            issue_gather(p, 2, 2)

            for s in range(_SUP):
                b = s % 5
                bn = (s + 3) % 5

                if s >= 2:
                    wait_scatter(bn)
                if s + 3 < _SUP:
                    issue_gather(p, s + 3, bn)
                wait_gather(b)
                scale(p, s, b)
                issue_scatter(p, s, b)

            wait_scatter((_SUP - 2) % 5)
            wait_scatter((_SUP - 1) % 5)

            @pl.when(sidx + 1 < _NSUP)
            def _():
                wait_prefetch(1 - p)

            return 0

        lax.fori_loop(0, _NSUP, super_body, 0)
        plsc.subcore_barrier()

        src = acc.at[pl.ds(rbase, _RPT)]

        @pl.when(cid == 0)
        def _():
            pltpu.sync_copy(src, out64.at[pl.ds(rbase, _RPT), pl.ds(0, _DH)])
            if want_halves:
                pltpu.sync_copy(src, out_lo.at[pl.ds(rbase, _RPT)])

        @pl.when(cid == 1)
        def _():
            pltpu.sync_copy(src, out64.at[pl.ds(rbase, _RPT), pl.ds(_DH, _DH)])
            if want_halves:
                pltpu.sync_copy(src, out_hi.at[pl.ds(rbase, _RPT)])

    return spmm


_make_spmm = functools.cache(_make_spmm)


def _make_user_gather():
    mesh = plsc.VectorSubcoreMesh(core_axis_name="c", subcore_axis_name="s")
    bpw = 1024 // (_NC * _NS)

    @functools.partial(
        pl.kernel,
        out_type=jax.ShapeDtypeStruct((1024, _D), jnp.float32),
        mesh=mesh,
        scratch_types=[
            pltpu.VMEM((bpw,), jnp.int32),
            pltpu.VMEM((bpw, _D), jnp.float32),
            pltpu.SemaphoreType.DMA,
        ],
        compiler_params=pltpu.CompilerParams(use_tc_tiling_on_sc=False),
    )
    def gather_k(table, idx, out, idx_v, rows_v, sem):
        wid = lax.axis_index("s") * _NC + lax.axis_index("c")
        base = wid * bpw
        pltpu.sync_copy(idx.at[pl.ds(base, bpw)], idx_v)
        pltpu.async_copy(table.at[idx_v], rows_v, sem).wait()
        pltpu.sync_copy(rows_v, out.at[pl.ds(base, bpw)])

    return gather_k


_make_user_gather = functools.cache(_make_user_gather)


_RB = 2000


def _split_body(ut_ref, it_ref, e64_ref, lo_ref, hi_ref):
    i = pl.program_id(0)
    x = jnp.where(jnp.full((1, 1), i < 5, jnp.bool_), ut_ref[...], it_ref[...])
    e64_ref[...] = x
    lo_ref[...] = x[:, :_DH]
    hi_ref[...] = x[:, _DH:]


def _split_tables(user_table, item_table):
    grid = _NN // _RB
    return pl.pallas_call(
        _split_body,
        grid=(grid,),
        in_specs=[
            pl.BlockSpec((_RB, _D), lambda i: (jnp.minimum(i, 4), 0)),
            pl.BlockSpec((_RB, _D), lambda i: (jnp.maximum(i - 5, 0), 0)),
        ],
        out_specs=[
            pl.BlockSpec((_RB, _D), lambda i: (i, 0)),
            pl.BlockSpec((_RB, _DH), lambda i: (i, 0)),
            pl.BlockSpec((_RB, _DH), lambda i: (i, 0)),
        ],
        out_shape=(jax.ShapeDtypeStruct((_NN, _D), jnp.float32),
                   jax.ShapeDtypeStruct((_NN, _DH), jnp.float32),
                   jax.ShapeDtypeStruct((_NN, _DH), jnp.float32)),
    )(user_table, item_table)


def _dense_body(e_ref, h_ref, wih_ref, whh_ref, bih_ref, bhh_ref, wg_ref,
                bg_ref, outlo_ref, outhi_ref):
    e = e_ref[...]
    h = h_ref[0, 0]
    gi = jnp.dot(e, wih_ref[...], preferred_element_type=jnp.float32) + bih_ref[...]
    gh = jnp.dot(h, whh_ref[...], preferred_element_type=jnp.float32) + bhh_ref[...]
    r = jax.nn.sigmoid(gi[:, :_D] + gh[:, :_D])
    z = jax.nn.sigmoid(gi[:, _D:2 * _D] + gh[:, _D:2 * _D])
    n = jnp.tanh(gi[:, 2 * _D:] + r * gh[:, 2 * _D:])
    gru = (1.0 - z) * n + z * h
    side = e * gru
    side = jnp.dot(side, wg_ref[...], preferred_element_type=jnp.float32) + bg_ref[...]
    x = side + e
    x = jnp.where(x >= 0.0, x, 0.2 * x)
    nrm = jnp.sqrt(jnp.sum(x * x, axis=1, keepdims=True))
    x = x / jnp.maximum(nrm, 1e-12)
    outlo_ref[...] = x[:, :_DH]
    outhi_ref[...] = x[:, _DH:]


def _dense_layer(e, h0, k, wih_t, whh_t, bih, bhh, wg, bg):
    grid = _NN // _RB
    return pl.pallas_call(
        _dense_body,
        grid=(grid,),
        in_specs=[
            pl.BlockSpec((_RB, _D), lambda i: (i, 0)),
            pl.BlockSpec((1, 1, _RB, _D), lambda i, k=k: (k, 0, i, 0)),
            pl.BlockSpec((_D, 3 * _D), lambda i: (0, 0)),
            pl.BlockSpec((_D, 3 * _D), lambda i: (0, 0)),
            pl.BlockSpec((1, 3 * _D), lambda i: (0, 0)),
            pl.BlockSpec((1, 3 * _D), lambda i: (0, 0)),
            pl.BlockSpec((_D, _D), lambda i: (0, 0)),
            pl.BlockSpec((1, _D), lambda i: (0, 0)),
        ],
        out_specs=[
            pl.BlockSpec((_RB, _DH), lambda i: (i, 0)),
            pl.BlockSpec((_RB, _DH), lambda i: (i, 0)),
        ],
        out_shape=(jax.ShapeDtypeStruct((_NN, _DH), jnp.float32),
                   jax.ShapeDtypeStruct((_NN, _DH), jnp.float32)),
    )(e, h0, wih_t, whh_t, bih, bhh, wg, bg)


def _combine_body(e0_ref, a1_ref, a2_ref, a3_ref, f1_ref, f2_ref, f3_ref,
                  out_ref, items_ref):
    i = pl.program_id(0)
    e0 = e0_ref[...]
    light1 = 0.25 * (e0 + a1_ref[...] + a2_ref[...] + a3_ref[...])
    light2 = e0 + f1_ref[...] + f2_ref[...] + f3_ref[...]
    res = light1 + _BETA * light2
    out_ref[...] = res

    @pl.when(i >= 5)
    def _():
        items_ref[...] = res


def _combine(e0, a1, a2, a3, f1, f2, f3):
    grid = _NN // _RB
    spec = pl.BlockSpec((_RB, _D), lambda i: (i, 0))
    return pl.pallas_call(
        _combine_body,
        grid=(grid,),
        in_specs=[spec] * 7,
        out_specs=[
            spec,
            pl.BlockSpec((_RB, _D), lambda i: (jnp.maximum(i - 5, 0), 0)),
        ],
        out_shape=(jax.ShapeDtypeStruct((_NN, _D), jnp.float32),
                   jax.ShapeDtypeStruct((_NI, _D), jnp.float32)),
    )(e0, a1, a2, a3, f1, f2, f3)


_IB = 2000


def _rating_body(it_ref, u_ref, out_ref):
    prod = lax.dot_general(it_ref[...], u_ref[...],
                           (((1,), (1,)), ((), ())),
                           preferred_element_type=jnp.float32)
    out_ref[...] = jax.nn.sigmoid(prod)


def _rating(u, items):
    grid = _NI // _IB
    out_t = pl.pallas_call(
        _rating_body,
        grid=(grid,),
        in_specs=[
            pl.BlockSpec((_IB, _D), lambda i: (i, 0)),
            pl.BlockSpec((1024, _D), lambda i: (0, 0)),
        ],
        out_specs=pl.BlockSpec((_IB, 1024), lambda i: (i, 0)),
        out_shape=jax.ShapeDtypeStruct((_NI, 1024), jnp.float32),
    )(items, u)
    return out_t.T


def kernel(users, edge_index, edge_values, user_table, item_table,
           w_ih, w_hh, b_ih, b_hh, W_gc, b_gc, h0):
    row = edge_index[0]
    col = edge_index[1]
    pad = _E_PAD - _E
    col_p = jnp.concatenate([col, jnp.zeros((pad,), col.dtype)])
    row_p = jnp.concatenate([row, jnp.zeros((pad,), row.dtype)])
    val_p = jnp.concatenate([edge_values, jnp.zeros((pad,), edge_values.dtype)])
    cols_g = col_p.reshape(_NGRP, _GRP)
    rows_g = row_p.reshape(_NGRP, _GRP)
    vals_g = val_p.reshape(_NGRP, _GRP)

    def spmm(emb_lo, emb_hi, want_halves):
        return _make_spmm(want_halves)(emb_lo, emb_hi, cols_g, rows_g, vals_g)

    e0_64, e0_lo, e0_hi = _split_tables(user_table, item_table)

    a1_64, a1_lo, a1_hi = spmm(e0_lo, e0_hi, True)
    a2_64, a2_lo, a2_hi = spmm(a1_lo, a1_hi, True)
    (a3_64,) = spmm(a2_lo, a2_hi, False)

    wih_t = w_ih.T
    whh_t = w_hh.T
    bih2 = b_ih.reshape(1, 3 * _D)
    bhh2 = b_hh.reshape(1, 3 * _D)
    f64 = e0_64
    f64s = []
    for k in range(_NLAYERS):
        g_lo, g_hi = _dense_layer(f64, h0, k, wih_t, whh_t, bih2, bhh2,
                                  W_gc[k], b_gc[k].reshape(1, _D))
        (f64,) = spmm(g_lo, g_hi, False)
        f64s.append(f64)

    comb, items = _combine(e0_64, a1_64, a2_64, a3_64, *f64s)

    u = _make_user_gather()(comb, users)
    return _rating(u, items)
